# baseline (speedup 1.0000x reference)
import functools

import jax
import jax.numpy as jnp
from jax import lax
from jax.experimental import pallas as pl
from jax.experimental.pallas import tpu as pltpu
from jax.experimental.pallas import tpu_sc as plsc

_BATCH = 16384
_NC = 1
_NS = 16
_NW = _NC * _NS
_PER_W = _BATCH // _NW


@functools.cache
def _build_gather_sc():
    mesh = plsc.VectorSubcoreMesh(core_axis_name="c", subcore_axis_name="s", num_cores=1)

    @functools.partial(
        pl.kernel,
        mesh=mesh,
        out_type=jax.ShapeDtypeStruct((_BATCH,), jnp.float32),
        scratch_types=[
            pltpu.VMEM((_PER_W,), jnp.int32),
            pltpu.VMEM((_PER_W,), jnp.float32),
            pltpu.SemaphoreType.DMA,
            pltpu.SemaphoreType.DMA,
            pltpu.SemaphoreType.DMA,
        ],
    )
    def _gather_sc(idx_hbm, table_hbm, out_hbm, idx_v, vals_v, isem, gsem, osem):
        wid = lax.axis_index("s") * _NC + lax.axis_index("c")
        base = wid * _PER_W
        nchunk = 4
        h = _PER_W // nchunk
        ic = [
            pltpu.async_copy(
                idx_hbm.at[pl.ds(base + k * h, h)], idx_v.at[pl.ds(k * h, h)], isem
            )
            for k in range(nchunk)
        ]
        gc = []
        for k in range(nchunk):
            ic[k].wait()
            gc.append(
                pltpu.async_copy(
                    table_hbm.at[idx_v.at[pl.ds(k * h, h)]],
                    vals_v.at[pl.ds(k * h, h)],
                    gsem,
                )
            )
        oc = []
        for k in range(nchunk):
            gc[k].wait()
            oc.append(
                pltpu.async_copy(
                    vals_v.at[pl.ds(k * h, h)], out_hbm.at[pl.ds(base + k * h, h)], osem
                )
            )
        for c in oc:
            c.wait()

    return _gather_sc


def kernel(id, map_param):
    return _build_gather_sc()(id.astype(jnp.int32), map_param)


# speedup vs baseline: 1.0429x; 1.0429x over previous
import functools

import jax
import jax.numpy as jnp
from jax import lax
from jax.experimental import pallas as pl
from jax.experimental.pallas import tpu as pltpu
from jax.experimental.pallas import tpu_sc as plsc

_BATCH = 16384
_NC = 1
_NS = 16
_NW = _NC * _NS
_PER_W = _BATCH // _NW


@functools.cache
def _build_gather_sc():
    mesh = plsc.VectorSubcoreMesh(core_axis_name="c", subcore_axis_name="s", num_cores=1)

    @functools.partial(
        pl.kernel,
        mesh=mesh,
        out_type=jax.ShapeDtypeStruct((_BATCH,), jnp.float32),
        scratch_types=[
            pltpu.VMEM((_PER_W,), jnp.int32),
            pltpu.VMEM((_PER_W,), jnp.float32),
            pltpu.SemaphoreType.DMA,
            pltpu.SemaphoreType.DMA,
            pltpu.SemaphoreType.DMA,
        ],
    )
    def _gather_sc(idx_hbm, table_hbm, out_hbm, idx_v, vals_v, isem, gsem, osem):
        wid = lax.axis_index("s") * _NC + lax.axis_index("c")
        base = wid * _PER_W
        nchunk = 2
        h = _PER_W // nchunk
        ic = [
            pltpu.async_copy(
                idx_hbm.at[pl.ds(base + k * h, h)], idx_v.at[pl.ds(k * h, h)], isem
            )
            for k in range(nchunk)
        ]
        gc = []
        for k in range(nchunk):
            ic[k].wait()
            gc.append(
                pltpu.async_copy(
                    table_hbm.at[idx_v.at[pl.ds(k * h, h)]],
                    vals_v.at[pl.ds(k * h, h)],
                    gsem,
                )
            )
        oc = []
        for k in range(nchunk):
            gc[k].wait()
            oc.append(
                pltpu.async_copy(
                    vals_v.at[pl.ds(k * h, h)], out_hbm.at[pl.ds(base + k * h, h)], osem
                )
            )
        for c in oc:
            c.wait()

    return _gather_sc


def kernel(id, map_param):
    return _build_gather_sc()(id.astype(jnp.int32), map_param)
